# baseline (speedup 1.0000x reference)
import jax
import jax.numpy as jnp
from jax.experimental import pallas as pl
from jax.experimental.pallas import tpu as pltpu

_B, _N, _S, _H = 4, 512, 64, 32
_CH = 8
_NBUF = 8
_T = (_B * _N) // _CH


def _body(nm_ref, x_hbm, w_ref, b_ref, o_hbm, xbuf, obuf, insem, outsem):
    def in_copy(t, slot):
        b = t // (_N // _CH)
        i0 = (t % (_N // _CH)) * _CH
        return pltpu.make_async_copy(
            x_hbm.at[b, pl.ds(i0, _CH)], xbuf.at[slot], insem.at[slot]
        )

    def out_copy(t, slot):
        b = t // (_N // _CH)
        i0 = (t % (_N // _CH)) * _CH
        return pltpu.make_async_copy(
            obuf.at[slot], o_hbm.at[b, pl.ds(i0, _CH)], outsem.at[slot]
        )

    for k in range(_NBUF):
        in_copy(k, k).start()

    def step(t, _):
        slot = jax.lax.rem(t, _NBUF)
        in_copy(t, slot).wait()
        x = xbuf[slot].reshape(_CH * _N, _S)
        y = jnp.dot(x, w_ref[...], preferred_element_type=jnp.float32)
        y = (y + b_ref[0]).reshape(_CH, _N, _H)
        @pl.when(t >= _NBUF)
        def _():
            out_copy(t - _NBUF, slot).wait()
        for c in range(_CH):
            obuf[slot, c] = y[c] * nm_ref[t * _CH + c]
        out_copy(t, slot).start()
        @pl.when(t + _NBUF < _T)
        def _():
            in_copy(t + _NBUF, slot).start()
        return 0

    jax.lax.fori_loop(0, _T, step, 0)
    for k in range(_NBUF):
        t = _T - _NBUF + k
        out_copy(t, jax.lax.rem(jnp.int32(t), _NBUF)).wait()


@jax.jit
def kernel(stacks, mask, W, b_lin):
    wt = W.T.astype(jnp.float32)
    bp = jnp.broadcast_to(b_lin.reshape(1, _H), (8, _H))
    nm = 1.0 - mask.reshape(-1).astype(jnp.float32)

    out = pl.pallas_call(
        _body,
        grid_spec=pltpu.PrefetchScalarGridSpec(
            num_scalar_prefetch=1,
            grid=(),
            in_specs=[
                pl.BlockSpec(memory_space=pltpu.MemorySpace.HBM),
                pl.BlockSpec(memory_space=pltpu.MemorySpace.VMEM),
                pl.BlockSpec(memory_space=pltpu.MemorySpace.VMEM),
            ],
            out_specs=pl.BlockSpec(memory_space=pltpu.MemorySpace.HBM),
            scratch_shapes=[
                pltpu.VMEM((_NBUF, _CH, _N, _S), jnp.float32),
                pltpu.VMEM((_NBUF, _CH, _N, _H), jnp.float32),
                pltpu.SemaphoreType.DMA((_NBUF,)),
                pltpu.SemaphoreType.DMA((_NBUF,)),
            ],
        ),
        out_shape=jax.ShapeDtypeStruct((_B, _N, _N, _H), jnp.float32),
    )(nm, stacks, wt, bp)
    return out
